# baseline (speedup 1.0000x reference)
import jax
import jax.numpy as jnp
from jax import lax
from jax.experimental import pallas as pl
from jax.experimental.pallas import tpu as pltpu
from jax.experimental.pallas import tpu_sc as plsc

N = 10000
IN_DIM = 128
HID = 64
HP = 128
E = 320000

_RB = 1000
_MB = 400


def _proj_body(x_ref, w_ref, b_ref, o_ref):
    o_ref[...] = (
        jnp.dot(x_ref[...], w_ref[...], preferred_element_type=jnp.float32)
        + b_ref[...]
    )


def _input_proj(x, W, b):
    grid = (N // _RB,)
    return pl.pallas_call(
        _proj_body,
        grid=grid,
        in_specs=[
            pl.BlockSpec((_RB, IN_DIM), lambda i: (i, 0)),
            pl.BlockSpec((IN_DIM, HP), lambda i: (0, 0)),
            pl.BlockSpec((1, HP), lambda i: (0, 0)),
        ],
        out_specs=pl.BlockSpec((_RB, HP), lambda i: (i, 0)),
        out_shape=jax.ShapeDtypeStruct((N, HP), jnp.float32),
        compiler_params=pltpu.CompilerParams(
            dimension_semantics=("parallel",),
        ),
    )(x, W, b.reshape(1, HP))


def _gin_mlp_body(h_ref, acc_ref, w1_ref, b1_ref, w2_ref, b2_ref, o_ref):
    z = acc_ref[0] + acc_ref[1] - h_ref[...]
    z = jax.nn.relu(
        jnp.dot(z, w1_ref[...], preferred_element_type=jnp.float32) + b1_ref[...]
    )
    o_ref[...] = (
        jnp.dot(z, w2_ref[...], preferred_element_type=jnp.float32) + b2_ref[...]
    )


def _gin_mlp(h, acc, W1, b1, W2, b2):
    grid = (N // _RB,)
    return pl.pallas_call(
        _gin_mlp_body,
        grid=grid,
        in_specs=[
            pl.BlockSpec((_RB, HP), lambda i: (i, 0)),
            pl.BlockSpec((2, _RB, HP), lambda i: (0, i, 0)),
            pl.BlockSpec((HP, HID), lambda i: (0, 0)),
            pl.BlockSpec((1, HID), lambda i: (0, 0)),
            pl.BlockSpec((HID, HP), lambda i: (0, 0)),
            pl.BlockSpec((1, HP), lambda i: (0, 0)),
        ],
        out_specs=pl.BlockSpec((_RB, HP), lambda i: (i, 0)),
        out_shape=jax.ShapeDtypeStruct((N, HP), jnp.float32),
        compiler_params=pltpu.CompilerParams(
            dimension_semantics=("parallel",),
        ),
    )(h, acc, W1, b1.reshape(1, HID), W2, b2.reshape(1, HP))


def _gram_body(a_ref, b_ref, o_ref):
    o_ref[...] = jax.lax.dot_general(
        a_ref[...],
        b_ref[...],
        (((1,), (1,)), ((), ())),
        preferred_element_type=jnp.float32,
    )


def _gram(emb):
    grid = (N // _MB,)
    return pl.pallas_call(
        _gram_body,
        grid=grid,
        in_specs=[
            pl.BlockSpec((_MB, HP), lambda i: (i, 0)),
            pl.BlockSpec((N, HP), lambda i: (0, 0)),
        ],
        out_specs=pl.BlockSpec((_MB, N), lambda i: (i, 0)),
        out_shape=jax.ShapeDtypeStruct((N, N), jnp.float32),
        compiler_params=pltpu.CompilerParams(
            dimension_semantics=("parallel",),
        ),
    )(emb, emb)


_NC = 2
_NS = 16
_SW = 128
_ET = E // (_NC * _NS)
_EP = 10240
_NPAD = _EP - _ET
_SR = _EP // _SW
_PH = 10
_PR = _SR // _PH
_AROWS = N + 128
_RT = 624
_RT_LAST = N - 15 * _RT


def _seg_body(tab_hbm, edge_hbm, out_hbm, acc_sh, src_v, dst_v, rows_a, rows_b, sem_a, sem_b):
    c = lax.axis_index("c")
    s = lax.axis_index("s")

    @pl.when(s < 15)
    def _():
        r0 = s * _RT
        pltpu.sync_copy(tab_hbm.at[pl.ds(r0, _RT)], acc_sh.at[pl.ds(r0, _RT)])

    @pl.when(s == 15)
    def _():
        r0 = 15 * _RT
        pltpu.sync_copy(tab_hbm.at[pl.ds(r0, _RT_LAST)], acc_sh.at[pl.ds(r0, _RT_LAST)])

    w = c * _NS + s
    plsc.subcore_barrier()

    def phase(p, carry):
        pltpu.sync_copy(edge_hbm.at[0, w, p], src_v)
        pltpu.sync_copy(edge_hbm.at[1, w, p], dst_v)
        pltpu.async_copy(tab_hbm.at[src_v.at[0]], rows_a, sem_a)

        def step(j, c2):
            nxt = j + 1

            @pl.when(jnp.logical_and(nxt < _PR, lax.rem(nxt, 2) == 1))
            def _():
                pltpu.async_copy(tab_hbm.at[src_v.at[nxt]], rows_b, sem_b)

            @pl.when(jnp.logical_and(nxt < _PR, lax.rem(nxt, 2) == 0))
            def _():
                pltpu.async_copy(tab_hbm.at[src_v.at[nxt]], rows_a, sem_a)

            @pl.when(lax.rem(j, 2) == 0)
            def _():
                pltpu.make_async_copy(tab_hbm.at[src_v.at[j]], rows_a, sem_a).wait()
                pltpu.sync_copy(rows_a, acc_sh.at[dst_v.at[j]], add=True)

            @pl.when(lax.rem(j, 2) == 1)
            def _():
                pltpu.make_async_copy(tab_hbm.at[src_v.at[j]], rows_b, sem_b).wait()
                pltpu.sync_copy(rows_b, acc_sh.at[dst_v.at[j]], add=True)

            return c2

        lax.fori_loop(0, _PR, step, 0)
        return carry

    lax.fori_loop(0, _PH, phase, 0)
    plsc.subcore_barrier()

    @pl.when(s < 15)
    def _():
        r0 = s * _RT
        pltpu.sync_copy(acc_sh.at[pl.ds(r0, _RT)], out_hbm.at[c, pl.ds(r0, _RT)])

    @pl.when(s == 15)
    def _():
        r0 = 15 * _RT
        pltpu.sync_copy(acc_sh.at[pl.ds(r0, _RT_LAST)], out_hbm.at[c, pl.ds(r0, _RT_LAST)])


_seg_sc = pl.kernel(
    _seg_body,
    out_type=jax.ShapeDtypeStruct((_NC, N, HP), jnp.float32),
    mesh=plsc.VectorSubcoreMesh(core_axis_name="c", subcore_axis_name="s"),
    scratch_types=[
        pltpu.VMEM_SHARED((_AROWS, HP), jnp.float32),
        pltpu.VMEM((_PR, _SW), jnp.int32),
        pltpu.VMEM((_PR, _SW), jnp.int32),
        pltpu.VMEM((_SW, HP), jnp.float32),
        pltpu.VMEM((_SW, HP), jnp.float32),
        pltpu.SemaphoreType.DMA,
        pltpu.SemaphoreType.DMA,
    ],
)


def kernel(x, edge_index, W_lin, b_lin, W_e1, b_e1, W_e2, b_e2, W_d1, b_d1, W_d2, b_d2):
    er = edge_index.reshape(2, _NC * _NS, _ET)
    pad_src = jnp.arange(_NPAD, dtype=jnp.int32) % N
    pad_dst = N + (jnp.arange(_NPAD, dtype=jnp.int32) % 128)
    pad = jnp.broadcast_to(
        jnp.stack([pad_src, pad_dst])[:, None, :], (2, _NC * _NS, _NPAD)
    )
    edges = jnp.concatenate([er, pad], axis=2).reshape(
        2, _NC * _NS, _PH, _PR, _SW
    )
    Wl = jnp.pad(W_lin, ((0, 0), (0, HP - HID)))
    bl = jnp.pad(b_lin, (0, HP - HID))
    We1 = jnp.pad(W_e1, ((0, HP - HID), (0, 0)))
    We2 = jnp.pad(W_e2, ((0, 0), (0, HP - HID)))
    be2 = jnp.pad(b_e2, (0, HP - HID))
    Wd1 = jnp.pad(W_d1, ((0, HP - HID), (0, 0)))

    h = _input_proj(x, Wl, bl)
    acc1 = _seg_sc(h, edges)
    emb = _gin_mlp(h, acc1, We1, b_e1, We2, be2)
    acc2 = _seg_sc(emb, edges)
    x_ = _gin_mlp(emb, acc2, Wd1, b_d1, W_d2, b_d2)
    s_ = _gram(emb)
    return (x_, s_)


# speedup vs baseline: 1.1604x; 1.1604x over previous
import jax
import jax.numpy as jnp
from jax import lax
from jax.experimental import pallas as pl
from jax.experimental.pallas import tpu as pltpu
from jax.experimental.pallas import tpu_sc as plsc

N = 10000
IN_DIM = 128
HID = 64
E = 320000

_RB = 1000
_MB = 400


def _proj_body(x_ref, w_ref, b_ref, o_ref):
    o_ref[...] = (
        jnp.dot(x_ref[...], w_ref[...], preferred_element_type=jnp.float32)
        + b_ref[...]
    )


def _input_proj(x, W, b):
    grid = (N // _RB,)
    return pl.pallas_call(
        _proj_body,
        grid=grid,
        in_specs=[
            pl.BlockSpec((_RB, IN_DIM), lambda i: (i, 0)),
            pl.BlockSpec((IN_DIM, HID), lambda i: (0, 0)),
            pl.BlockSpec((1, HID), lambda i: (0, 0)),
        ],
        out_specs=pl.BlockSpec((_RB, HID), lambda i: (i, 0)),
        out_shape=jax.ShapeDtypeStruct((N, HID), jnp.float32),
        compiler_params=pltpu.CompilerParams(
            dimension_semantics=("parallel",),
        ),
    )(x, W, b.reshape(1, HID))


def _gin_mlp_body(h_ref, acc_ref, w1_ref, b1_ref, w2_ref, b2_ref, o_ref):
    z = acc_ref[0] + acc_ref[1] - h_ref[...]
    z = jax.nn.relu(
        jnp.dot(z, w1_ref[...], preferred_element_type=jnp.float32) + b1_ref[...]
    )
    o_ref[...] = (
        jnp.dot(z, w2_ref[...], preferred_element_type=jnp.float32) + b2_ref[...]
    )


def _gin_mlp(h, acc, W1, b1, W2, b2, out_dim):
    grid = (N // _RB,)
    return pl.pallas_call(
        _gin_mlp_body,
        grid=grid,
        in_specs=[
            pl.BlockSpec((_RB, HID), lambda i: (i, 0)),
            pl.BlockSpec((2, _RB, HID), lambda i: (0, i, 0)),
            pl.BlockSpec((HID, HID), lambda i: (0, 0)),
            pl.BlockSpec((1, HID), lambda i: (0, 0)),
            pl.BlockSpec((HID, out_dim), lambda i: (0, 0)),
            pl.BlockSpec((1, out_dim), lambda i: (0, 0)),
        ],
        out_specs=pl.BlockSpec((_RB, out_dim), lambda i: (i, 0)),
        out_shape=jax.ShapeDtypeStruct((N, out_dim), jnp.float32),
        compiler_params=pltpu.CompilerParams(
            dimension_semantics=("parallel",),
        ),
    )(h, acc, W1, b1.reshape(1, HID), W2, b2.reshape(1, out_dim))


def _gram_body(a_ref, b_ref, o_ref):
    o_ref[...] = jax.lax.dot_general(
        a_ref[...],
        b_ref[...],
        (((1,), (1,)), ((), ())),
        preferred_element_type=jnp.float32,
    )


def _gram(emb):
    grid = (N // _MB,)
    return pl.pallas_call(
        _gram_body,
        grid=grid,
        in_specs=[
            pl.BlockSpec((_MB, HID), lambda i: (i, 0)),
            pl.BlockSpec((N, HID), lambda i: (0, 0)),
        ],
        out_specs=pl.BlockSpec((_MB, N), lambda i: (i, 0)),
        out_shape=jax.ShapeDtypeStruct((N, N), jnp.float32),
        compiler_params=pltpu.CompilerParams(
            dimension_semantics=("parallel",),
        ),
    )(emb, emb)


_NC = 2
_NS = 16
_SW = 80
_ET = E // (_NC * _NS)
_SR = _ET // _SW
_PH = 5
_PR = _SR // _PH
_RT = 624
_RT_LAST = N - 15 * _RT


def _seg_body(tab_hbm, edge_hbm, out_hbm, acc_sh, src_v, dst_v, rows_a, rows_b, sem_a, sem_b):
    c = lax.axis_index("c")
    s = lax.axis_index("s")

    @pl.when(s < 15)
    def _():
        r0 = s * _RT
        pltpu.sync_copy(tab_hbm.at[pl.ds(r0, _RT)], acc_sh.at[pl.ds(r0, _RT)])

    @pl.when(s == 15)
    def _():
        r0 = 15 * _RT
        pltpu.sync_copy(tab_hbm.at[pl.ds(r0, _RT_LAST)], acc_sh.at[pl.ds(r0, _RT_LAST)])

    w = c * _NS + s
    plsc.subcore_barrier()

    def phase(p, carry):
        pltpu.sync_copy(edge_hbm.at[0, w, p], src_v)
        pltpu.sync_copy(edge_hbm.at[1, w, p], dst_v)
        pltpu.async_copy(tab_hbm.at[src_v.at[0]], rows_a, sem_a)

        def step(j, c2):
            nxt = j + 1

            @pl.when(jnp.logical_and(nxt < _PR, lax.rem(nxt, 2) == 1))
            def _():
                pltpu.async_copy(tab_hbm.at[src_v.at[nxt]], rows_b, sem_b)

            @pl.when(jnp.logical_and(nxt < _PR, lax.rem(nxt, 2) == 0))
            def _():
                pltpu.async_copy(tab_hbm.at[src_v.at[nxt]], rows_a, sem_a)

            @pl.when(lax.rem(j, 2) == 0)
            def _():
                pltpu.make_async_copy(tab_hbm.at[src_v.at[j]], rows_a, sem_a).wait()
                pltpu.sync_copy(rows_a, acc_sh.at[dst_v.at[j]], add=True)

            @pl.when(lax.rem(j, 2) == 1)
            def _():
                pltpu.make_async_copy(tab_hbm.at[src_v.at[j]], rows_b, sem_b).wait()
                pltpu.sync_copy(rows_b, acc_sh.at[dst_v.at[j]], add=True)

            return c2

        lax.fori_loop(0, _PR, step, 0)
        return carry

    lax.fori_loop(0, _PH, phase, 0)
    plsc.subcore_barrier()

    @pl.when(s < 15)
    def _():
        r0 = s * _RT
        pltpu.sync_copy(acc_sh.at[pl.ds(r0, _RT)], out_hbm.at[c, pl.ds(r0, _RT)])

    @pl.when(s == 15)
    def _():
        r0 = 15 * _RT
        pltpu.sync_copy(acc_sh.at[pl.ds(r0, _RT_LAST)], out_hbm.at[c, pl.ds(r0, _RT_LAST)])


_seg_sc = pl.kernel(
    _seg_body,
    out_type=jax.ShapeDtypeStruct((_NC, N, HID), jnp.float32),
    mesh=plsc.VectorSubcoreMesh(core_axis_name="c", subcore_axis_name="s"),
    scratch_types=[
        pltpu.VMEM_SHARED((N, HID), jnp.float32),
        pltpu.VMEM((_PR, _SW), jnp.int32),
        pltpu.VMEM((_PR, _SW), jnp.int32),
        pltpu.VMEM((_SW, HID), jnp.float32),
        pltpu.VMEM((_SW, HID), jnp.float32),
        pltpu.SemaphoreType.DMA,
        pltpu.SemaphoreType.DMA,
    ],
    compiler_params=pltpu.CompilerParams(use_tc_tiling_on_sc=False),
)


def kernel(x, edge_index, W_lin, b_lin, W_e1, b_e1, W_e2, b_e2, W_d1, b_d1, W_d2, b_d2):
    edges = edge_index.reshape(2, _NC * _NS, _PH, _PR, _SW)
    h = _input_proj(x, W_lin, b_lin)
    acc1 = _seg_sc(h, edges)
    emb = _gin_mlp(h, acc1, W_e1, b_e1, W_e2, b_e2, HID)
    acc2 = _seg_sc(emb, edges)
    x_ = _gin_mlp(emb, acc2, W_d1, b_d1, W_d2, b_d2, IN_DIM)
    s_ = _gram(emb)
    return (x_, s_)


# speedup vs baseline: 1.3303x; 1.1464x over previous
import jax
import jax.numpy as jnp
from jax import lax
from jax.experimental import pallas as pl
from jax.experimental.pallas import tpu as pltpu
from jax.experimental.pallas import tpu_sc as plsc

N = 10000
IN_DIM = 128
HID = 64
E = 320000

_RB = 1000
_MB = 400


def _proj_body(x_ref, w_ref, b_ref, o_ref):
    o_ref[...] = (
        jnp.dot(x_ref[...], w_ref[...], preferred_element_type=jnp.float32)
        + b_ref[...]
    )


def _input_proj(x, W, b):
    grid = (N // _RB,)
    return pl.pallas_call(
        _proj_body,
        grid=grid,
        in_specs=[
            pl.BlockSpec((_RB, IN_DIM), lambda i: (i, 0)),
            pl.BlockSpec((IN_DIM, HID), lambda i: (0, 0)),
            pl.BlockSpec((1, HID), lambda i: (0, 0)),
        ],
        out_specs=pl.BlockSpec((_RB, HID), lambda i: (i, 0)),
        out_shape=jax.ShapeDtypeStruct((N, HID), jnp.float32),
        compiler_params=pltpu.CompilerParams(
            dimension_semantics=("parallel",),
        ),
    )(x, W, b.reshape(1, HID))


def _gin_mlp_body(h_ref, acc_ref, w1_ref, b1_ref, w2_ref, b2_ref, o_ref):
    z = acc_ref[0] + acc_ref[1] - h_ref[...]
    z = jax.nn.relu(
        jnp.dot(z, w1_ref[...], preferred_element_type=jnp.float32) + b1_ref[...]
    )
    o_ref[...] = (
        jnp.dot(z, w2_ref[...], preferred_element_type=jnp.float32) + b2_ref[...]
    )


def _gin_mlp(h, acc, W1, b1, W2, b2, out_dim):
    grid = (N // _RB,)
    return pl.pallas_call(
        _gin_mlp_body,
        grid=grid,
        in_specs=[
            pl.BlockSpec((_RB, HID), lambda i: (i, 0)),
            pl.BlockSpec((2, _RB, HID), lambda i: (0, i, 0)),
            pl.BlockSpec((HID, HID), lambda i: (0, 0)),
            pl.BlockSpec((1, HID), lambda i: (0, 0)),
            pl.BlockSpec((HID, out_dim), lambda i: (0, 0)),
            pl.BlockSpec((1, out_dim), lambda i: (0, 0)),
        ],
        out_specs=pl.BlockSpec((_RB, out_dim), lambda i: (i, 0)),
        out_shape=jax.ShapeDtypeStruct((N, out_dim), jnp.float32),
        compiler_params=pltpu.CompilerParams(
            dimension_semantics=("parallel",),
        ),
    )(h, acc, W1, b1.reshape(1, HID), W2, b2.reshape(1, out_dim))


def _gram_body(a_ref, b_ref, o_ref):
    o_ref[...] = jax.lax.dot_general(
        a_ref[...],
        b_ref[...],
        (((1,), (1,)), ((), ())),
        preferred_element_type=jnp.float32,
    )


def _gram(emb):
    grid = (N // _MB,)
    return pl.pallas_call(
        _gram_body,
        grid=grid,
        in_specs=[
            pl.BlockSpec((_MB, HID), lambda i: (i, 0)),
            pl.BlockSpec((N, HID), lambda i: (0, 0)),
        ],
        out_specs=pl.BlockSpec((_MB, N), lambda i: (i, 0)),
        out_shape=jax.ShapeDtypeStruct((N, N), jnp.float32),
        compiler_params=pltpu.CompilerParams(
            dimension_semantics=("parallel",),
        ),
    )(emb, emb)


_NC = 2
_NS = 16
_SW = 80
_ET = E // (_NC * _NS)
_SR = _ET // _SW
_PH = 5
_PR = _SR // _PH
_RT = 624
_RT_LAST = N - 15 * _RT


def _seg_body(tab_hbm, edge_hbm, out_hbm, tab_sh, acc_sh, src_v, dst_v, rows_a, rows_b, sem_a, sem_b):
    c = lax.axis_index("c")
    s = lax.axis_index("s")

    @pl.when(s < 15)
    def _():
        r0 = s * _RT
        pltpu.sync_copy(tab_hbm.at[pl.ds(r0, _RT)], tab_sh.at[pl.ds(r0, _RT)])
        pltpu.sync_copy(tab_hbm.at[pl.ds(r0, _RT)], acc_sh.at[pl.ds(r0, _RT)])

    @pl.when(s == 15)
    def _():
        r0 = 15 * _RT
        pltpu.sync_copy(tab_hbm.at[pl.ds(r0, _RT_LAST)], tab_sh.at[pl.ds(r0, _RT_LAST)])
        pltpu.sync_copy(tab_hbm.at[pl.ds(r0, _RT_LAST)], acc_sh.at[pl.ds(r0, _RT_LAST)])

    w = c * _NS + s
    plsc.subcore_barrier()

    def phase(p, carry):
        pltpu.sync_copy(edge_hbm.at[0, w, p], src_v)
        pltpu.sync_copy(edge_hbm.at[1, w, p], dst_v)
        pltpu.async_copy(tab_sh.at[src_v.at[0]], rows_a, sem_a)

        def step(j, c2):
            nxt = j + 1

            @pl.when(jnp.logical_and(nxt < _PR, lax.rem(nxt, 2) == 1))
            def _():
                pltpu.async_copy(tab_sh.at[src_v.at[nxt]], rows_b, sem_b)

            @pl.when(jnp.logical_and(nxt < _PR, lax.rem(nxt, 2) == 0))
            def _():
                pltpu.async_copy(tab_sh.at[src_v.at[nxt]], rows_a, sem_a)

            @pl.when(lax.rem(j, 2) == 0)
            def _():
                pltpu.make_async_copy(tab_sh.at[src_v.at[j]], rows_a, sem_a).wait()
                pltpu.sync_copy(rows_a, acc_sh.at[dst_v.at[j]], add=True)

            @pl.when(lax.rem(j, 2) == 1)
            def _():
                pltpu.make_async_copy(tab_sh.at[src_v.at[j]], rows_b, sem_b).wait()
                pltpu.sync_copy(rows_b, acc_sh.at[dst_v.at[j]], add=True)

            return c2

        lax.fori_loop(0, _PR, step, 0)
        return carry

    lax.fori_loop(0, _PH, phase, 0)
    plsc.subcore_barrier()

    @pl.when(s < 15)
    def _():
        r0 = s * _RT
        pltpu.sync_copy(acc_sh.at[pl.ds(r0, _RT)], out_hbm.at[c, pl.ds(r0, _RT)])

    @pl.when(s == 15)
    def _():
        r0 = 15 * _RT
        pltpu.sync_copy(acc_sh.at[pl.ds(r0, _RT_LAST)], out_hbm.at[c, pl.ds(r0, _RT_LAST)])


_seg_sc = pl.kernel(
    _seg_body,
    out_type=jax.ShapeDtypeStruct((_NC, N, HID), jnp.float32),
    mesh=plsc.VectorSubcoreMesh(core_axis_name="c", subcore_axis_name="s"),
    scratch_types=[
        pltpu.VMEM_SHARED((N, HID), jnp.float32),
        pltpu.VMEM_SHARED((N, HID), jnp.float32),
        pltpu.VMEM((_PR, _SW), jnp.int32),
        pltpu.VMEM((_PR, _SW), jnp.int32),
        pltpu.VMEM((_SW, HID), jnp.float32),
        pltpu.VMEM((_SW, HID), jnp.float32),
        pltpu.SemaphoreType.DMA,
        pltpu.SemaphoreType.DMA,
    ],
    compiler_params=pltpu.CompilerParams(use_tc_tiling_on_sc=False),
)


def kernel(x, edge_index, W_lin, b_lin, W_e1, b_e1, W_e2, b_e2, W_d1, b_d1, W_d2, b_d2):
    edges = edge_index.reshape(2, _NC * _NS, _PH, _PR, _SW)
    h = _input_proj(x, W_lin, b_lin)
    acc1 = _seg_sc(h, edges)
    emb = _gin_mlp(h, acc1, W_e1, b_e1, W_e2, b_e2, HID)
    acc2 = _seg_sc(emb, edges)
    x_ = _gin_mlp(emb, acc2, W_d1, b_d1, W_d2, b_d2, IN_DIM)
    s_ = _gram(emb)
    return (x_, s_)


# speedup vs baseline: 1.3690x; 1.0291x over previous
import jax
import jax.numpy as jnp
from jax import lax
from jax.experimental import pallas as pl
from jax.experimental.pallas import tpu as pltpu
from jax.experimental.pallas import tpu_sc as plsc

N = 10000
IN_DIM = 128
HID = 64
E = 320000

_RB = 2000
_MB = 400


def _proj_body(x_ref, w_ref, b_ref, o_ref):
    o_ref[...] = (
        jnp.dot(x_ref[...], w_ref[...], preferred_element_type=jnp.float32)
        + b_ref[...]
    )


def _input_proj(x, W, b):
    grid = (N // _RB,)
    return pl.pallas_call(
        _proj_body,
        grid=grid,
        in_specs=[
            pl.BlockSpec((_RB, IN_DIM), lambda i: (i, 0)),
            pl.BlockSpec((IN_DIM, HID), lambda i: (0, 0)),
            pl.BlockSpec((1, HID), lambda i: (0, 0)),
        ],
        out_specs=pl.BlockSpec((_RB, HID), lambda i: (i, 0)),
        out_shape=jax.ShapeDtypeStruct((N, HID), jnp.float32),
        compiler_params=pltpu.CompilerParams(
            dimension_semantics=("parallel",),
        ),
    )(x, W, b.reshape(1, HID))


def _gin_mlp_body(h_ref, acc_ref, w1_ref, b1_ref, w2_ref, b2_ref, o_ref):
    z = acc_ref[0] + acc_ref[1] - h_ref[...]
    z = jax.nn.relu(
        jnp.dot(z, w1_ref[...], preferred_element_type=jnp.float32) + b1_ref[...]
    )
    o_ref[...] = (
        jnp.dot(z, w2_ref[...], preferred_element_type=jnp.float32) + b2_ref[...]
    )


def _gin_mlp(h, acc, W1, b1, W2, b2, out_dim):
    grid = (N // _RB,)
    return pl.pallas_call(
        _gin_mlp_body,
        grid=grid,
        in_specs=[
            pl.BlockSpec((_RB, HID), lambda i: (i, 0)),
            pl.BlockSpec((2, _RB, HID), lambda i: (0, i, 0)),
            pl.BlockSpec((HID, HID), lambda i: (0, 0)),
            pl.BlockSpec((1, HID), lambda i: (0, 0)),
            pl.BlockSpec((HID, out_dim), lambda i: (0, 0)),
            pl.BlockSpec((1, out_dim), lambda i: (0, 0)),
        ],
        out_specs=pl.BlockSpec((_RB, out_dim), lambda i: (i, 0)),
        out_shape=jax.ShapeDtypeStruct((N, out_dim), jnp.float32),
        compiler_params=pltpu.CompilerParams(
            dimension_semantics=("parallel",),
        ),
    )(h, acc, W1, b1.reshape(1, HID), W2, b2.reshape(1, out_dim))


def _gram_body(a_ref, b_ref, o_ref):
    o_ref[...] = jax.lax.dot_general(
        a_ref[...],
        b_ref[...],
        (((1,), (1,)), ((), ())),
        preferred_element_type=jnp.float32,
    )


def _gram(emb):
    grid = (N // _MB,)
    return pl.pallas_call(
        _gram_body,
        grid=grid,
        in_specs=[
            pl.BlockSpec((_MB, HID), lambda i: (i, 0)),
            pl.BlockSpec((N, HID), lambda i: (0, 0)),
        ],
        out_specs=pl.BlockSpec((_MB, N), lambda i: (i, 0)),
        out_shape=jax.ShapeDtypeStruct((N, N), jnp.float32),
        compiler_params=pltpu.CompilerParams(
            dimension_semantics=("parallel",),
        ),
    )(emb, emb)


_NC = 2
_NS = 16
_SW = 80
_ET = E // (_NC * _NS)
_SR = _ET // _SW
_PH = 5
_PR = _SR // _PH
_RT = 624
_RT_LAST = N - 15 * _RT


def _seg_body(tab_hbm, edge_hbm, out_hbm, tab_sh, acc_sh, src_v, dst_v, rows_a, rows_b, sem_a, sem_b, sem_sa, sem_sb):
    c = lax.axis_index("c")
    s = lax.axis_index("s")

    @pl.when(s < 15)
    def _():
        r0 = s * _RT
        pltpu.sync_copy(tab_hbm.at[pl.ds(r0, _RT)], tab_sh.at[pl.ds(r0, _RT)])
        pltpu.sync_copy(tab_hbm.at[pl.ds(r0, _RT)], acc_sh.at[pl.ds(r0, _RT)])

    @pl.when(s == 15)
    def _():
        r0 = 15 * _RT
        pltpu.sync_copy(tab_hbm.at[pl.ds(r0, _RT_LAST)], tab_sh.at[pl.ds(r0, _RT_LAST)])
        pltpu.sync_copy(tab_hbm.at[pl.ds(r0, _RT_LAST)], acc_sh.at[pl.ds(r0, _RT_LAST)])

    w = c * _NS + s
    plsc.subcore_barrier()

    def phase(p, carry):
        pltpu.sync_copy(edge_hbm.at[0, w, p], src_v)
        pltpu.sync_copy(edge_hbm.at[1, w, p], dst_v)
        pltpu.async_copy(tab_sh.at[src_v.at[0]], rows_a, sem_a)

        def step(j, c2):
            nxt = j + 1
            prv = j - 1
            in_n = nxt < _PR

            @pl.when(jnp.logical_and(in_n, jnp.logical_and(prv >= 0, lax.rem(nxt, 2) == 0)))
            def _():
                pltpu.make_async_copy(rows_a, acc_sh.at[dst_v.at[prv]], sem_sa).wait()

            @pl.when(jnp.logical_and(in_n, jnp.logical_and(prv >= 0, lax.rem(nxt, 2) == 1)))
            def _():
                pltpu.make_async_copy(rows_b, acc_sh.at[dst_v.at[prv]], sem_sb).wait()

            @pl.when(jnp.logical_and(in_n, lax.rem(nxt, 2) == 0))
            def _():
                pltpu.async_copy(tab_sh.at[src_v.at[nxt]], rows_a, sem_a)

            @pl.when(jnp.logical_and(in_n, lax.rem(nxt, 2) == 1))
            def _():
                pltpu.async_copy(tab_sh.at[src_v.at[nxt]], rows_b, sem_b)

            @pl.when(lax.rem(j, 2) == 0)
            def _():
                pltpu.make_async_copy(tab_sh.at[src_v.at[j]], rows_a, sem_a).wait()
                pltpu.async_copy(rows_a, acc_sh.at[dst_v.at[j]], sem_sa, add=True)

            @pl.when(lax.rem(j, 2) == 1)
            def _():
                pltpu.make_async_copy(tab_sh.at[src_v.at[j]], rows_b, sem_b).wait()
                pltpu.async_copy(rows_b, acc_sh.at[dst_v.at[j]], sem_sb, add=True)

            return c2

        lax.fori_loop(0, _PR, step, 0)
        pltpu.make_async_copy(rows_b, acc_sh.at[dst_v.at[_PR - 2]], sem_sb).wait()
        pltpu.make_async_copy(rows_a, acc_sh.at[dst_v.at[_PR - 1]], sem_sa).wait()
        return carry

    lax.fori_loop(0, _PH, phase, 0)
    plsc.subcore_barrier()

    @pl.when(s < 15)
    def _():
        r0 = s * _RT
        pltpu.sync_copy(acc_sh.at[pl.ds(r0, _RT)], out_hbm.at[c, pl.ds(r0, _RT)])

    @pl.when(s == 15)
    def _():
        r0 = 15 * _RT
        pltpu.sync_copy(acc_sh.at[pl.ds(r0, _RT_LAST)], out_hbm.at[c, pl.ds(r0, _RT_LAST)])


_seg_sc = pl.kernel(
    _seg_body,
    out_type=jax.ShapeDtypeStruct((_NC, N, HID), jnp.float32),
    mesh=plsc.VectorSubcoreMesh(core_axis_name="c", subcore_axis_name="s"),
    scratch_types=[
        pltpu.VMEM_SHARED((N, HID), jnp.float32),
        pltpu.VMEM_SHARED((N, HID), jnp.float32),
        pltpu.VMEM((_PR, _SW), jnp.int32),
        pltpu.VMEM((_PR, _SW), jnp.int32),
        pltpu.VMEM((_SW, HID), jnp.float32),
        pltpu.VMEM((_SW, HID), jnp.float32),
        pltpu.SemaphoreType.DMA,
        pltpu.SemaphoreType.DMA,
        pltpu.SemaphoreType.DMA,
        pltpu.SemaphoreType.DMA,
    ],
    compiler_params=pltpu.CompilerParams(use_tc_tiling_on_sc=False),
)


def kernel(x, edge_index, W_lin, b_lin, W_e1, b_e1, W_e2, b_e2, W_d1, b_d1, W_d2, b_d2):
    edges = edge_index.reshape(2, _NC * _NS, _PH, _PR, _SW)
    h = _input_proj(x, W_lin, b_lin)
    acc1 = _seg_sc(h, edges)
    emb = _gin_mlp(h, acc1, W_e1, b_e1, W_e2, b_e2, HID)
    acc2 = _seg_sc(emb, edges)
    x_ = _gin_mlp(emb, acc2, W_d1, b_d1, W_d2, b_d2, IN_DIM)
    s_ = _gram(emb)
    return (x_, s_)
